# Initial kernel scaffold; baseline (speedup 1.0000x reference)
import jax
import jax.numpy as jnp
from jax.experimental import pallas as pl


def kernel(indices, offsets, Wq, Wr):
    raise NotImplementedError("write your pallas kernel here")


# speedup vs baseline: 189.7723x; 189.7723x over previous
import jax
import jax.numpy as jnp
from jax import lax
from jax.experimental import pallas as pl
from jax.experimental.pallas import tpu as pltpu
from jax.experimental.pallas import tpu_sc as plsc

QF = 1024
L = 16
NC, NS = 2, 16
NW = NC * NS
WIN = 512
STAGE = 8
D = 64


def _body(n_idx, n_bags, idx_hbm, off_hbm, wq_hbm, wr_hbm, out_hbm,
          wq_v, wr_v, off_v, off2_v, win_v, stage_v):
    bags_per_w = n_bags // NW
    c = lax.axis_index("c")
    s = lax.axis_index("s")
    w = s * NC + c
    b0 = pl.multiple_of(w * bags_per_w, bags_per_w)

    pltpu.sync_copy(wq_hbm, wq_v)
    pltpu.sync_copy(wr_hbm, wr_v)
    pltpu.sync_copy(off_hbm.at[pl.ds(b0, bags_per_w)],
                    off_v.at[pl.ds(0, bags_per_w)])

    @pl.when(w < NW - 1)
    def _():
        pltpu.sync_copy(off_hbm.at[pl.ds(b0 + bags_per_w, 8)],
                        off2_v.at[pl.ds(0, 8)])

    i_end_w = jnp.where(w < NW - 1, off2_v[pl.ds(0, L)][0], n_idx)
    zero = jnp.zeros((L,), jnp.float32)

    def bag_body(b, win_base):
        s_i = off_v[pl.ds(b, L)][0]
        nxt = off_v[pl.ds(b + 1, L)][0]
        e_i = jnp.where(b < bags_per_w - 1, nxt, i_end_w)

        def ibody(i, st):
            wb, a0, a1, a2, a3 = st
            need = i >= wb + WIN
            nwb = pl.multiple_of(
                jnp.where(need,
                          jnp.minimum(lax.bitwise_and(i, -8), n_idx - WIN),
                          wb), 8)

            @pl.when(need)
            def _():
                pltpu.sync_copy(idx_hbm.at[pl.ds(nwb, WIN)],
                                win_v.at[pl.ds(0, WIN)])

            ix = win_v[pl.ds(i - nwb, L)][0]
            qb = lax.shift_left(lax.shift_right_logical(ix, 10), 6)
            rb = lax.shift_left(lax.bitwise_and(ix, QF - 1), 6)
            a0 = a0 + (wq_v[pl.ds(qb + 0 * L, L)] + wr_v[pl.ds(rb + 0 * L, L)])
            a1 = a1 + (wq_v[pl.ds(qb + 1 * L, L)] + wr_v[pl.ds(rb + 1 * L, L)])
            a2 = a2 + (wq_v[pl.ds(qb + 2 * L, L)] + wr_v[pl.ds(rb + 2 * L, L)])
            a3 = a3 + (wq_v[pl.ds(qb + 3 * L, L)] + wr_v[pl.ds(rb + 3 * L, L)])
            return (nwb, a0, a1, a2, a3)

        st = lax.fori_loop(s_i, e_i, ibody,
                           (win_base, zero, zero, zero, zero))
        win_base, a0, a1, a2, a3 = st

        sb = lax.bitwise_and(b, STAGE - 1)
        stage_v[sb, pl.ds(0 * L, L)] = a0
        stage_v[sb, pl.ds(1 * L, L)] = a1
        stage_v[sb, pl.ds(2 * L, L)] = a2
        stage_v[sb, pl.ds(3 * L, L)] = a3

        @pl.when(sb == STAGE - 1)
        def _():
            dst = pl.multiple_of(b0 + b - (STAGE - 1), STAGE)
            pltpu.sync_copy(stage_v, out_hbm.at[pl.ds(dst, STAGE)])

        return win_base

    lax.fori_loop(0, bags_per_w, bag_body, jnp.int32(-(1 << 30)))


def kernel(indices, offsets, Wq, Wr):
    n_idx = indices.shape[0]
    n_bags = offsets.shape[0]
    qn, d = Wq.shape
    qf = Wr.shape[0]
    indices = indices.astype(jnp.int32)
    offsets = offsets.astype(jnp.int32)
    wq_flat = Wq.reshape(-1)
    wr_flat = Wr.reshape(-1)

    mesh = plsc.VectorSubcoreMesh(core_axis_name="c", subcore_axis_name="s")
    body = lambda *refs: _body(n_idx, n_bags, *refs)
    fn = pl.kernel(
        body,
        out_type=jax.ShapeDtypeStruct((n_bags, d), jnp.float32),
        mesh=mesh,
        scratch_types=[
            pltpu.VMEM((qn * d,), jnp.float32),
            pltpu.VMEM((qf * d,), jnp.float32),
            pltpu.VMEM((n_bags // NW + 24, ), jnp.int32),
            pltpu.VMEM((16,), jnp.int32),
            pltpu.VMEM((WIN + 16,), jnp.int32),
            pltpu.VMEM((STAGE, d), jnp.float32),
        ],
    )
    return fn(indices, offsets, wq_flat, wr_flat)


# speedup vs baseline: 362.3304x; 1.9093x over previous
import jax
import jax.numpy as jnp
from jax import lax
from jax.experimental import pallas as pl
from jax.experimental.pallas import tpu as pltpu
from jax.experimental.pallas import tpu_sc as plsc

QF = 1024
L = 16
NC, NS = 2, 16
NW = NC * NS
WIN = 512
STAGE = 8
D = 64


def _body(n_idx, n_bags, idx_hbm, off_hbm, wq_hbm, wr_hbm, out_hbm,
          wq_v, wr_v, off_v, off2_v, win_v, stage_v):
    bags_per_w = n_bags // NW
    c = lax.axis_index("c")
    s = lax.axis_index("s")
    w = s * NC + c
    b0 = pl.multiple_of(w * bags_per_w, bags_per_w)

    pltpu.sync_copy(wq_hbm, wq_v)
    pltpu.sync_copy(wr_hbm, wr_v)
    pltpu.sync_copy(off_hbm.at[pl.ds(b0, bags_per_w)],
                    off_v.at[pl.ds(0, bags_per_w)])

    @pl.when(w < NW - 1)
    def _():
        pltpu.sync_copy(off_hbm.at[pl.ds(b0 + bags_per_w, 8)],
                        off2_v.at[pl.ds(0, 8)])

    i_end_w = jnp.where(w < NW - 1, off2_v[pl.ds(0, L)][0], n_idx)
    zero = jnp.zeros((L,), jnp.float32)

    def bag_body(b, win_base):
        s_i = off_v[pl.ds(b, L)][0]
        nxt = off_v[pl.ds(b + 1, L)][0]
        e_i = jnp.where(b < bags_per_w - 1, nxt, i_end_w)
        n4_end = s_i + lax.bitwise_and(e_i - s_i, -4)

        def gbody(g, st):
            wb, a0, a1, a2, a3 = st
            i = s_i + lax.shift_left(g, 2)
            need = i + 4 > wb + WIN
            nwb = pl.multiple_of(
                jnp.where(need,
                          jnp.minimum(lax.bitwise_and(i, -8), n_idx - WIN),
                          wb), 8)

            @pl.when(need)
            def _():
                pltpu.sync_copy(idx_hbm.at[pl.ds(nwb, WIN)],
                                win_v.at[pl.ds(0, WIN)])

            v = win_v[pl.ds(i - nwb, L)]
            rows = []
            for j in range(4):
                ix = v[j]
                qb = lax.shift_left(lax.shift_right_logical(ix, 10), 6)
                rb = lax.shift_left(lax.bitwise_and(ix, QF - 1), 6)
                rows.append([
                    wq_v[pl.ds(qb + k * L, L)] + wr_v[pl.ds(rb + k * L, L)]
                    for k in range(4)])
            a0 = a0 + ((rows[0][0] + rows[1][0]) + (rows[2][0] + rows[3][0]))
            a1 = a1 + ((rows[0][1] + rows[1][1]) + (rows[2][1] + rows[3][1]))
            a2 = a2 + ((rows[0][2] + rows[1][2]) + (rows[2][2] + rows[3][2]))
            a3 = a3 + ((rows[0][3] + rows[1][3]) + (rows[2][3] + rows[3][3]))
            return (nwb, a0, a1, a2, a3)

        def ibody(i, st):
            wb, a0, a1, a2, a3 = st
            need = i >= wb + WIN
            nwb = pl.multiple_of(
                jnp.where(need,
                          jnp.minimum(lax.bitwise_and(i, -8), n_idx - WIN),
                          wb), 8)

            @pl.when(need)
            def _():
                pltpu.sync_copy(idx_hbm.at[pl.ds(nwb, WIN)],
                                win_v.at[pl.ds(0, WIN)])

            ix = win_v[pl.ds(i - nwb, L)][0]
            qb = lax.shift_left(lax.shift_right_logical(ix, 10), 6)
            rb = lax.shift_left(lax.bitwise_and(ix, QF - 1), 6)
            a0 = a0 + (wq_v[pl.ds(qb + 0 * L, L)] + wr_v[pl.ds(rb + 0 * L, L)])
            a1 = a1 + (wq_v[pl.ds(qb + 1 * L, L)] + wr_v[pl.ds(rb + 1 * L, L)])
            a2 = a2 + (wq_v[pl.ds(qb + 2 * L, L)] + wr_v[pl.ds(rb + 2 * L, L)])
            a3 = a3 + (wq_v[pl.ds(qb + 3 * L, L)] + wr_v[pl.ds(rb + 3 * L, L)])
            return (nwb, a0, a1, a2, a3)

        st = lax.fori_loop(0, lax.shift_right_logical(n4_end - s_i, 2),
                           gbody, (win_base, zero, zero, zero, zero))
        st = lax.fori_loop(n4_end, e_i, ibody, st)
        win_base, a0, a1, a2, a3 = st

        sb = lax.bitwise_and(b, STAGE - 1)
        stage_v[sb, pl.ds(0 * L, L)] = a0
        stage_v[sb, pl.ds(1 * L, L)] = a1
        stage_v[sb, pl.ds(2 * L, L)] = a2
        stage_v[sb, pl.ds(3 * L, L)] = a3

        @pl.when(sb == STAGE - 1)
        def _():
            dst = pl.multiple_of(b0 + b - (STAGE - 1), STAGE)
            pltpu.sync_copy(stage_v, out_hbm.at[pl.ds(dst, STAGE)])

        return win_base

    lax.fori_loop(0, bags_per_w, bag_body, jnp.int32(-(1 << 30)))


def kernel(indices, offsets, Wq, Wr):
    n_idx = indices.shape[0]
    n_bags = offsets.shape[0]
    qn, d = Wq.shape
    qf = Wr.shape[0]
    indices = indices.astype(jnp.int32)
    offsets = offsets.astype(jnp.int32)
    wq_flat = Wq.reshape(-1)
    wr_flat = Wr.reshape(-1)

    mesh = plsc.VectorSubcoreMesh(core_axis_name="c", subcore_axis_name="s")
    body = lambda *refs: _body(n_idx, n_bags, *refs)
    fn = pl.kernel(
        body,
        out_type=jax.ShapeDtypeStruct((n_bags, d), jnp.float32),
        mesh=mesh,
        scratch_types=[
            pltpu.VMEM((qn * d,), jnp.float32),
            pltpu.VMEM((qf * d,), jnp.float32),
            pltpu.VMEM((n_bags // NW + 24, ), jnp.int32),
            pltpu.VMEM((16,), jnp.int32),
            pltpu.VMEM((WIN + 16,), jnp.int32),
            pltpu.VMEM((STAGE, d), jnp.float32),
        ],
    )
    return fn(indices, offsets, wq_flat, wr_flat)


# speedup vs baseline: 364.5210x; 1.0060x over previous
import jax
import jax.numpy as jnp
from jax import lax
from jax.experimental import pallas as pl
from jax.experimental.pallas import tpu as pltpu
from jax.experimental.pallas import tpu_sc as plsc

QF = 1024
L = 16
NC, NS = 2, 16
NW = NC * NS
WIN = 512
STAGE = 8
D = 64


def _body(n_idx, n_bags, idx_hbm, off_hbm, wq_hbm, wr_hbm, out_hbm,
          wq_v, wr_v, off_v, off2_v, win_v, stage_v):
    bags_per_w = n_bags // NW
    c = lax.axis_index("c")
    s = lax.axis_index("s")
    w = s * NC + c
    b0 = pl.multiple_of(w * bags_per_w, bags_per_w)

    pltpu.sync_copy(wq_hbm, wq_v)
    pltpu.sync_copy(wr_hbm, wr_v)
    pltpu.sync_copy(off_hbm.at[pl.ds(b0, bags_per_w)],
                    off_v.at[pl.ds(0, bags_per_w)])

    @pl.when(w < NW - 1)
    def _():
        pltpu.sync_copy(off_hbm.at[pl.ds(b0 + bags_per_w, 8)],
                        off2_v.at[pl.ds(0, 8)])

    i_end_w = jnp.where(w < NW - 1, off2_v[pl.ds(0, L)][0], n_idx)
    zero = jnp.zeros((L,), jnp.float32)

    def bag_body(b, win_base):
        s_i = off_v[pl.ds(b, L)][0]
        nxt = off_v[pl.ds(b + 1, L)][0]
        e_i = jnp.where(b < bags_per_w - 1, nxt, i_end_w)
        n4_end = s_i + lax.bitwise_and(e_i - s_i, -8)

        def gbody(g, st):
            wb, a0, a1, a2, a3 = st
            i = s_i + lax.shift_left(g, 3)
            need = i + 8 > wb + WIN
            nwb = pl.multiple_of(
                jnp.where(need,
                          jnp.minimum(lax.bitwise_and(i, -8), n_idx - WIN),
                          wb), 8)

            @pl.when(need)
            def _():
                pltpu.sync_copy(idx_hbm.at[pl.ds(nwb, WIN)],
                                win_v.at[pl.ds(0, WIN)])

            v = win_v[pl.ds(i - nwb, L)]
            rows = []
            for j in range(8):
                ix = v[j]
                qb = lax.shift_left(lax.shift_right_logical(ix, 10), 6)
                rb = lax.shift_left(lax.bitwise_and(ix, QF - 1), 6)
                rows.append([
                    wq_v[pl.ds(qb + k * L, L)] + wr_v[pl.ds(rb + k * L, L)]
                    for k in range(4)])
            accs = [a0, a1, a2, a3]
            for k in range(4):
                t01 = rows[0][k] + rows[1][k]
                t23 = rows[2][k] + rows[3][k]
                t45 = rows[4][k] + rows[5][k]
                t67 = rows[6][k] + rows[7][k]
                accs[k] = accs[k] + ((t01 + t23) + (t45 + t67))
            a0, a1, a2, a3 = accs
            return (nwb, a0, a1, a2, a3)

        def ibody(i, st):
            wb, a0, a1, a2, a3 = st
            need = i >= wb + WIN
            nwb = pl.multiple_of(
                jnp.where(need,
                          jnp.minimum(lax.bitwise_and(i, -8), n_idx - WIN),
                          wb), 8)

            @pl.when(need)
            def _():
                pltpu.sync_copy(idx_hbm.at[pl.ds(nwb, WIN)],
                                win_v.at[pl.ds(0, WIN)])

            ix = win_v[pl.ds(i - nwb, L)][0]
            qb = lax.shift_left(lax.shift_right_logical(ix, 10), 6)
            rb = lax.shift_left(lax.bitwise_and(ix, QF - 1), 6)
            a0 = a0 + (wq_v[pl.ds(qb + 0 * L, L)] + wr_v[pl.ds(rb + 0 * L, L)])
            a1 = a1 + (wq_v[pl.ds(qb + 1 * L, L)] + wr_v[pl.ds(rb + 1 * L, L)])
            a2 = a2 + (wq_v[pl.ds(qb + 2 * L, L)] + wr_v[pl.ds(rb + 2 * L, L)])
            a3 = a3 + (wq_v[pl.ds(qb + 3 * L, L)] + wr_v[pl.ds(rb + 3 * L, L)])
            return (nwb, a0, a1, a2, a3)

        st = lax.fori_loop(0, lax.shift_right_logical(n4_end - s_i, 3),
                           gbody, (win_base, zero, zero, zero, zero))
        st = lax.fori_loop(n4_end, e_i, ibody, st)
        win_base, a0, a1, a2, a3 = st

        sb = lax.bitwise_and(b, STAGE - 1)
        stage_v[sb, pl.ds(0 * L, L)] = a0
        stage_v[sb, pl.ds(1 * L, L)] = a1
        stage_v[sb, pl.ds(2 * L, L)] = a2
        stage_v[sb, pl.ds(3 * L, L)] = a3

        @pl.when(sb == STAGE - 1)
        def _():
            dst = pl.multiple_of(b0 + b - (STAGE - 1), STAGE)
            pltpu.sync_copy(stage_v, out_hbm.at[pl.ds(dst, STAGE)])

        return win_base

    lax.fori_loop(0, bags_per_w, bag_body, jnp.int32(-(1 << 30)))


def kernel(indices, offsets, Wq, Wr):
    n_idx = indices.shape[0]
    n_bags = offsets.shape[0]
    qn, d = Wq.shape
    qf = Wr.shape[0]
    indices = indices.astype(jnp.int32)
    offsets = offsets.astype(jnp.int32)
    wq_flat = Wq.reshape(-1)
    wr_flat = Wr.reshape(-1)

    mesh = plsc.VectorSubcoreMesh(core_axis_name="c", subcore_axis_name="s")
    body = lambda *refs: _body(n_idx, n_bags, *refs)
    fn = pl.kernel(
        body,
        out_type=jax.ShapeDtypeStruct((n_bags, d), jnp.float32),
        mesh=mesh,
        scratch_types=[
            pltpu.VMEM((qn * d,), jnp.float32),
            pltpu.VMEM((qf * d,), jnp.float32),
            pltpu.VMEM((n_bags // NW + 24, ), jnp.int32),
            pltpu.VMEM((16,), jnp.int32),
            pltpu.VMEM((WIN + 16,), jnp.int32),
            pltpu.VMEM((STAGE, d), jnp.float32),
        ],
    )
    return fn(indices, offsets, wq_flat, wr_flat)
